# Initial kernel scaffold; baseline (speedup 1.0000x reference)
import jax
import jax.numpy as jnp
from jax.experimental import pallas as pl


def kernel(hidden_states, ln1_w, ln2_w, Wq, Wk, Wv, Wo, Wr, Wg, Wu, Wd):
    raise NotImplementedError("write your pallas kernel here")


# speedup vs baseline: 1.2402x; 1.2402x over previous
import functools

import jax
import jax.numpy as jnp
from jax.experimental import pallas as pl

B, S, D, H, F, E, K = 1, 2048, 1024, 16, 4096, 8, 2
HD = D // H
EPS = 1e-6

BS = 256
BQ = 256
FT = 512
SB = S // BS


def _rms_qkv_body(x_ref, w_ref, wq_ref, wk_ref, wv_ref, q_ref, k_ref, v_ref):
    x = x_ref[...]
    h = x * jax.lax.rsqrt(jnp.mean(x * x, axis=1, keepdims=True) + EPS) * w_ref[...]
    q_ref[...] = jnp.dot(h, wq_ref[...], preferred_element_type=jnp.float32)
    k_ref[...] = jnp.dot(h, wk_ref[...], preferred_element_type=jnp.float32)
    v_ref[...] = jnp.dot(h, wv_ref[...], preferred_element_type=jnp.float32)


def _rope(x, cos, sin):
    x1 = x[:, : HD // 2]
    x2 = x[:, HD // 2:]
    rot = jnp.concatenate([-x2, x1], axis=1)
    return x * cos + rot * sin


def _attn_body(q_ref, k_ref, v_ref, cq_ref, sq_ref, ck_ref, sk_ref, o_ref):
    q = _rope(q_ref[0], cq_ref[...], sq_ref[...]) * (HD ** -0.5)
    k = _rope(k_ref[0], ck_ref[...], sk_ref[...])
    s = jax.lax.dot_general(q, k, (((1,), (1,)), ((), ())),
                            preferred_element_type=jnp.float32)
    m = jnp.max(s, axis=1, keepdims=True)
    e = jnp.exp(s - m)
    num = jnp.dot(e, v_ref[0], preferred_element_type=jnp.float32)
    o_ref[0] = num / jnp.sum(e, axis=1, keepdims=True)


def _router_body(ao_ref, x_ref, wo_ref, w2_ref, wr_ref,
                 x2_ref, h2_ref, wi_ref, us_ref, ps_ref, lb_ref):
    sb = pl.program_id(0)
    x2 = x_ref[...] + jnp.dot(ao_ref[...], wo_ref[...],
                              preferred_element_type=jnp.float32)
    x2_ref[...] = x2
    h2 = x2 * jax.lax.rsqrt(jnp.mean(x2 * x2, axis=1, keepdims=True) + EPS) * w2_ref[...]
    h2_ref[...] = h2
    logits = jnp.dot(h2, wr_ref[...], preferred_element_type=jnp.float32)
    lmax = jnp.max(logits, axis=1, keepdims=True)
    el = jnp.exp(logits - lmax)
    probs = el / jnp.sum(el, axis=1, keepdims=True)
    ids = jax.lax.broadcasted_iota(jnp.int32, (BS, E), 1)
    m1 = jnp.max(probs, axis=1, keepdims=True)
    a1 = jnp.min(jnp.where(probs == m1, ids, E), axis=1, keepdims=True)
    oh1 = (ids == a1).astype(jnp.float32)
    probs2 = jnp.where(ids == a1, -1.0, probs)
    m2 = jnp.max(probs2, axis=1, keepdims=True)
    a2 = jnp.min(jnp.where(probs2 == m2, ids, E), axis=1, keepdims=True)
    oh2 = (ids == a2).astype(jnp.float32)
    wsum = m1 + m2
    wi_ref[...] = (oh1 * m1 + oh2 * m2) / wsum

    @pl.when(sb == 0)
    def _init():
        us_ref[...] = jnp.zeros_like(us_ref)
        ps_ref[...] = jnp.zeros_like(ps_ref)
        lb_ref[...] = jnp.zeros_like(lb_ref)

    us_ref[...] += jnp.sum(oh1 + oh2, axis=0, keepdims=True)
    ps_ref[...] += jnp.sum(probs, axis=0, keepdims=True)

    @pl.when(sb == SB - 1)
    def _fin():
        lb_ref[...] = jnp.sum(us_ref[...] * ps_ref[...], axis=1, keepdims=True) \
            * (float(E) / (S * float(S)))


def _moe_body(h2_ref, x2_ref, wi_ref, wg_ref, wu_ref, wd_ref, o_ref):
    e = pl.program_id(0)
    f = pl.program_id(1)

    @pl.when((e == 0) & (f == 0))
    def _init():
        o_ref[...] = x2_ref[...]

    h2 = h2_ref[...]
    g = jax.nn.silu(jnp.dot(h2, wg_ref[0], preferred_element_type=jnp.float32))
    u = jnp.dot(h2, wu_ref[0], preferred_element_type=jnp.float32)
    eo = jnp.dot(g * u, wd_ref[0], preferred_element_type=jnp.float32)
    o_ref[...] += eo * wi_ref[0]


def kernel(hidden_states, ln1_w, ln2_w, Wq, Wk, Wv, Wo, Wr, Wg, Wu, Wd):
    x = hidden_states.reshape(S, D)
    w1 = ln1_w.reshape(1, D)
    w2 = ln2_w.reshape(1, D)

    inv_freq = 1.0 / (10000.0 ** (jnp.arange(0, HD, 2, dtype=jnp.float32) / HD))
    t = jnp.arange(S, dtype=jnp.float32)
    freqs = t[:, None] * inv_freq[None, :]
    emb = jnp.concatenate([freqs, freqs], axis=-1)
    cos = jnp.cos(emb)
    sin = jnp.sin(emb)

    q, k, v = pl.pallas_call(
        _rms_qkv_body,
        grid=(SB,),
        in_specs=[
            pl.BlockSpec((BS, D), lambda i: (i, 0)),
            pl.BlockSpec((1, D), lambda i: (0, 0)),
            pl.BlockSpec((D, D), lambda i: (0, 0)),
            pl.BlockSpec((D, D), lambda i: (0, 0)),
            pl.BlockSpec((D, D), lambda i: (0, 0)),
        ],
        out_specs=[pl.BlockSpec((BS, D), lambda i: (i, 0))] * 3,
        out_shape=[jax.ShapeDtypeStruct((S, D), jnp.float32)] * 3,
    )(x, w1, Wq, Wk, Wv)

    qh = q.reshape(S, H, HD).transpose(1, 0, 2)
    kh = k.reshape(S, H, HD).transpose(1, 0, 2)
    vh = v.reshape(S, H, HD).transpose(1, 0, 2)

    aoh = pl.pallas_call(
        _attn_body,
        grid=(H, S // BQ),
        in_specs=[
            pl.BlockSpec((1, BQ, HD), lambda h, i: (h, i, 0)),
            pl.BlockSpec((1, S, HD), lambda h, i: (h, 0, 0)),
            pl.BlockSpec((1, S, HD), lambda h, i: (h, 0, 0)),
            pl.BlockSpec((BQ, HD), lambda h, i: (i, 0)),
            pl.BlockSpec((BQ, HD), lambda h, i: (i, 0)),
            pl.BlockSpec((S, HD), lambda h, i: (0, 0)),
            pl.BlockSpec((S, HD), lambda h, i: (0, 0)),
        ],
        out_specs=pl.BlockSpec((1, BQ, HD), lambda h, i: (h, i, 0)),
        out_shape=jax.ShapeDtypeStruct((H, S, HD), jnp.float32),
    )(qh, kh, vh, cos, sin, cos, sin)
    ao = aoh.transpose(1, 0, 2).reshape(S, D)

    x2, h2, wi, _us, _ps, lb = pl.pallas_call(
        _router_body,
        grid=(SB,),
        in_specs=[
            pl.BlockSpec((BS, D), lambda i: (i, 0)),
            pl.BlockSpec((BS, D), lambda i: (i, 0)),
            pl.BlockSpec((D, D), lambda i: (0, 0)),
            pl.BlockSpec((1, D), lambda i: (0, 0)),
            pl.BlockSpec((D, E), lambda i: (0, 0)),
        ],
        out_specs=[
            pl.BlockSpec((BS, D), lambda i: (i, 0)),
            pl.BlockSpec((BS, D), lambda i: (i, 0)),
            pl.BlockSpec((BS, E), lambda i: (i, 0)),
            pl.BlockSpec((1, E), lambda i: (0, 0)),
            pl.BlockSpec((1, E), lambda i: (0, 0)),
            pl.BlockSpec((1, 1), lambda i: (0, 0)),
        ],
        out_shape=[
            jax.ShapeDtypeStruct((S, D), jnp.float32),
            jax.ShapeDtypeStruct((S, D), jnp.float32),
            jax.ShapeDtypeStruct((S, E), jnp.float32),
            jax.ShapeDtypeStruct((1, E), jnp.float32),
            jax.ShapeDtypeStruct((1, E), jnp.float32),
            jax.ShapeDtypeStruct((1, 1), jnp.float32),
        ],
    )(ao, x, Wo, w2, Wr)

    wiT = wi.T.reshape(E, S, 1)

    out = pl.pallas_call(
        _moe_body,
        grid=(E, F // FT),
        in_specs=[
            pl.BlockSpec((S, D), lambda e, f: (0, 0)),
            pl.BlockSpec((S, D), lambda e, f: (0, 0)),
            pl.BlockSpec((1, S, 1), lambda e, f: (e, 0, 0)),
            pl.BlockSpec((1, D, FT), lambda e, f: (e, 0, f)),
            pl.BlockSpec((1, D, FT), lambda e, f: (e, 0, f)),
            pl.BlockSpec((1, FT, D), lambda e, f: (e, f, 0)),
        ],
        out_specs=pl.BlockSpec((S, D), lambda e, f: (0, 0)),
        out_shape=jax.ShapeDtypeStruct((S, D), jnp.float32),
    )(h2, x2, wiT, Wg, Wu, Wd)

    return (out.reshape(B, S, D), lb.reshape(()))


# speedup vs baseline: 1.2832x; 1.0347x over previous
import functools

import jax
import jax.numpy as jnp
from jax import lax
from jax.experimental import pallas as pl
from jax.experimental.pallas import tpu as pltpu
from jax.experimental.pallas import tpu_sc as plsc

B, S, D, H, F, E, K = 1, 2048, 1024, 16, 4096, 8, 2
HD = D // H
EPS = 1e-6

BS = 256
BQ = 256
SB = S // BS

BLK = 128
NB = 40
PAD = NB * BLK
NBP = 48
FT = 1024
FB = F // FT

NW = 32
GCH = 80
LANE = 16


def _rms_qkv_body(x_ref, w_ref, wq_ref, wk_ref, wv_ref, q_ref, k_ref, v_ref):
    x = x_ref[...]
    h = x * lax.rsqrt(jnp.mean(x * x, axis=1, keepdims=True) + EPS) * w_ref[...]
    q_ref[...] = jnp.dot(h, wq_ref[...], preferred_element_type=jnp.float32)
    k_ref[...] = jnp.dot(h, wk_ref[...], preferred_element_type=jnp.float32)
    v_ref[...] = jnp.dot(h, wv_ref[...], preferred_element_type=jnp.float32)


def _rope(x, cos, sin):
    x1 = x[:, : HD // 2]
    x2 = x[:, HD // 2:]
    rot = jnp.concatenate([-x2, x1], axis=1)
    return x * cos + rot * sin


def _attn_body(q_ref, k_ref, v_ref, cq_ref, sq_ref, ck_ref, sk_ref, o_ref):
    q = _rope(q_ref[0], cq_ref[...], sq_ref[...]) * (HD ** -0.5)
    k = _rope(k_ref[0], ck_ref[...], sk_ref[...])
    s = lax.dot_general(q, k, (((1,), (1,)), ((), ())),
                        preferred_element_type=jnp.float32)
    m = jnp.max(s, axis=1, keepdims=True)
    e = jnp.exp(s - m)
    num = jnp.dot(e, v_ref[0], preferred_element_type=jnp.float32)
    o_ref[0] = num / jnp.sum(e, axis=1, keepdims=True)


def _router_body(ao_ref, x_ref, wo_ref, w2_ref, wr_ref,
                 x2_ref, h2_ref, a1_ref, a2_ref, w1_ref, w2o_ref,
                 us_ref, ps_ref, lb_ref):
    sb = pl.program_id(0)
    x2 = x_ref[...] + jnp.dot(ao_ref[...], wo_ref[...],
                              preferred_element_type=jnp.float32)
    x2_ref[...] = x2
    h2 = x2 * lax.rsqrt(jnp.mean(x2 * x2, axis=1, keepdims=True) + EPS) * w2_ref[...]
    h2_ref[...] = h2
    logits = jnp.dot(h2, wr_ref[...], preferred_element_type=jnp.float32)
    lmax = jnp.max(logits, axis=1, keepdims=True)
    el = jnp.exp(logits - lmax)
    probs = el / jnp.sum(el, axis=1, keepdims=True)
    ids = lax.broadcasted_iota(jnp.int32, (BS, E), 1)
    m1 = jnp.max(probs, axis=1, keepdims=True)
    a1 = jnp.min(jnp.where(probs == m1, ids, E), axis=1, keepdims=True)
    oh1 = (ids == a1).astype(jnp.float32)
    probs2 = jnp.where(ids == a1, -1.0, probs)
    m2 = jnp.max(probs2, axis=1, keepdims=True)
    a2 = jnp.min(jnp.where(probs2 == m2, ids, E), axis=1, keepdims=True)
    oh2 = (ids == a2).astype(jnp.float32)
    wsum = m1 + m2
    a1_ref[...] = a1
    a2_ref[...] = a2
    w1_ref[...] = m1 / wsum
    w2o_ref[...] = m2 / wsum

    @pl.when(sb == 0)
    def _init():
        us_ref[...] = jnp.zeros_like(us_ref)
        ps_ref[...] = jnp.zeros_like(ps_ref)
        lb_ref[...] = jnp.zeros_like(lb_ref)

    us_ref[...] += jnp.sum(oh1 + oh2, axis=0, keepdims=True)
    ps_ref[...] += jnp.sum(probs, axis=0, keepdims=True)

    @pl.when(sb == SB - 1)
    def _fin():
        lb_ref[...] = jnp.sum(us_ref[...] * ps_ref[...], axis=1, keepdims=True) \
            * (float(E) / (S * float(S)))


_SC_MESH = plsc.VectorSubcoreMesh(core_axis_name="c", subcore_axis_name="s")
NCH = S // LANE


@functools.partial(
    pl.kernel,
    mesh=_SC_MESH,
    compiler_params=pltpu.CompilerParams(needs_layout_passes=False),
    out_type=[
        jax.ShapeDtypeStruct((PAD,), jnp.int32),
        jax.ShapeDtypeStruct((S,), jnp.int32),
        jax.ShapeDtypeStruct((S,), jnp.int32),
        jax.ShapeDtypeStruct((NBP,), jnp.int32),
    ],
    scratch_types=[
        pltpu.VMEM((S,), jnp.int32),
        pltpu.VMEM((S,), jnp.int32),
        pltpu.VMEM((S,), jnp.int32),
        pltpu.VMEM((S,), jnp.int32),
        pltpu.VMEM((PAD,), jnp.int32),
        pltpu.VMEM((S,), jnp.int32),
        pltpu.VMEM((S,), jnp.int32),
        pltpu.VMEM((NBP,), jnp.int32),
        pltpu.VMEM((LANE,), jnp.int32),
        pltpu.VMEM((LANE,), jnp.int32),
    ],
)
def _sc_plan(eiA_hbm, eiB_hbm, srcrow_hbm, posA_hbm, posB_hbm, bexp_hbm,
             eA_v, eB_v, rankA_v, rankB_v, srcrow_v, posA_v, posB_v, bexp_v,
             cnt_v, off_v):
    wid = lax.axis_index("s") * 2 + lax.axis_index("c")

    @pl.when(wid == 0)
    def _():
        pltpu.sync_copy(eiA_hbm, eA_v)
        pltpu.sync_copy(eiB_hbm, eB_v)
        lane = lax.iota(jnp.int32, LANE)
        cnt_v[...] = jnp.zeros((LANE,), jnp.int32)

        def rank_pass(src_v, dst_v):
            def body(c, _):
                ch = src_v[pl.ds(c * LANE, LANE)]
                cnt = cnt_v[...]
                rank = jnp.zeros((LANE,), jnp.int32)
                for e in range(E):
                    m = ch == e
                    mi = jnp.where(m, 1, 0)
                    cs = plsc.cumsum(mi)
                    cnt_e = jnp.sum(jnp.where(lane == e, cnt, 0))
                    rank = jnp.where(m, cnt_e + cs - 1, rank)
                    tote = jnp.sum(mi)
                    cnt = jnp.where(lane == e, cnt + tote, cnt)
                dst_v[pl.ds(c * LANE, LANE)] = rank
                cnt_v[...] = cnt
                return 0
            lax.fori_loop(0, NCH, body, 0)

        rank_pass(eA_v, rankA_v)
        rank_pass(eB_v, rankB_v)

        cnt = cnt_v[...]
        blocks = lax.shift_right_logical(cnt + (BLK - 1), 7)
        cumblk = plsc.cumsum(blocks)
        off_v[...] = (cumblk - blocks) * BLK

        def zero_body(i, _):
            srcrow_v[pl.ds(i * LANE, LANE)] = jnp.zeros((LANE,), jnp.int32)
            return 0
        lax.fori_loop(0, PAD // LANE, zero_body, 0)

        def pos_pass(src_v, rank_v, pos_v):
            def body(c, _):
                ch = src_v[pl.ds(c * LANE, LANE)]
                offv = off_v[...]
                off = jnp.zeros((LANE,), jnp.int32)
                for e in range(E):
                    off_e = jnp.sum(jnp.where(lane == e, offv, 0))
                    off = jnp.where(ch == e, off_e, off)
                pos = off + rank_v[pl.ds(c * LANE, LANE)]
                pos_v[pl.ds(c * LANE, LANE)] = pos
                tok = lane + c * LANE
                plsc.store_scatter(srcrow_v, [pos], tok)
                return 0
            lax.fori_loop(0, NCH, body, 0)

        pos_pass(eA_v, rankA_v, posA_v)
        pos_pass(eB_v, rankB_v, posB_v)

        for cc in range(NBP // LANE):
            nb = lane + cc * LANE
            be = jnp.zeros((LANE,), jnp.int32)
            for e in range(E - 1):
                ce = jnp.sum(jnp.where(lane == e, cumblk, 0))
                be = be + jnp.where(nb >= ce, 1, 0)
            bexp_v[pl.ds(cc * LANE, LANE)] = be

        pltpu.sync_copy(srcrow_v, srcrow_hbm)
        pltpu.sync_copy(posA_v, posA_hbm)
        pltpu.sync_copy(posB_v, posB_hbm)
        pltpu.sync_copy(bexp_v, bexp_hbm)


@functools.partial(
    pl.kernel,
    mesh=_SC_MESH,
    compiler_params=pltpu.CompilerParams(needs_layout_passes=False),
    out_type=jax.ShapeDtypeStruct((PAD, D), jnp.float32),
    scratch_types=[
        pltpu.VMEM((GCH,), jnp.int32),
        pltpu.VMEM((GCH, D), jnp.float32),
        pltpu.SemaphoreType.DMA,
    ],
)
def _sc_gather(h2_hbm, srcrow_hbm, xg_hbm, idx_v, rows_v, sem):
    wid = lax.axis_index("s") * 2 + lax.axis_index("c")
    for j in range(PAD // NW // GCH):
        base = wid * (PAD // NW) + j * GCH
        pltpu.sync_copy(srcrow_hbm.at[pl.ds(base, GCH)], idx_v)
        pltpu.async_copy(h2_hbm.at[idx_v], rows_v, sem).wait()
        pltpu.sync_copy(rows_v, xg_hbm.at[pl.ds(base, GCH)])


CCH = S // NW


@functools.partial(
    pl.kernel,
    mesh=_SC_MESH,
    compiler_params=pltpu.CompilerParams(needs_layout_passes=False),
    out_type=[
        jax.ShapeDtypeStruct((S, D), jnp.float32),
        jax.ShapeDtypeStruct((S, D), jnp.float32),
    ],
    scratch_types=[
        pltpu.VMEM((CCH,), jnp.int32),
        pltpu.VMEM((CCH, D), jnp.float32),
        pltpu.SemaphoreType.DMA,
    ],
)
def _sc_combine_gather(og_hbm, posA_hbm, posB_hbm, gA_hbm, gB_hbm,
                       idx_v, rows_v, sem):
    wid = lax.axis_index("s") * 2 + lax.axis_index("c")
    base = wid * CCH
    pltpu.sync_copy(posA_hbm.at[pl.ds(base, CCH)], idx_v)
    pltpu.async_copy(og_hbm.at[idx_v], rows_v, sem).wait()
    pltpu.sync_copy(rows_v, gA_hbm.at[pl.ds(base, CCH)])
    pltpu.sync_copy(posB_hbm.at[pl.ds(base, CCH)], idx_v)
    pltpu.async_copy(og_hbm.at[idx_v], rows_v, sem).wait()
    pltpu.sync_copy(rows_v, gB_hbm.at[pl.ds(base, CCH)])


def _gmm_body(bexp_ref, xg_ref, wg_ref, wu_ref, wd_ref, og_ref, acc_ref):
    f = pl.program_id(0)
    nb = pl.program_id(1)
    xb = xg_ref[...]
    g = jax.nn.silu(jnp.dot(xb, wg_ref[0], preferred_element_type=jnp.float32))
    u = jnp.dot(xb, wu_ref[0], preferred_element_type=jnp.float32)
    contrib = jnp.dot(g * u, wd_ref[0], preferred_element_type=jnp.float32)
    sl = pl.ds(nb * BLK, BLK)

    @pl.when(f == 0)
    def _first():
        acc_ref[sl, :] = contrib

    @pl.when(f > 0)
    def _rest():
        acc_ref[sl, :] += contrib

    @pl.when(f == FB - 1)
    def _write():
        og_ref[...] = acc_ref[sl, :]


def _combine_body(x2_ref, ga_ref, gb_ref, w1_ref, w2_ref, o_ref):
    o_ref[...] = x2_ref[...] + w1_ref[...] * ga_ref[...] + w2_ref[...] * gb_ref[...]


def kernel(hidden_states, ln1_w, ln2_w, Wq, Wk, Wv, Wo, Wr, Wg, Wu, Wd):
    x = hidden_states.reshape(S, D)
    w1 = ln1_w.reshape(1, D)
    w2 = ln2_w.reshape(1, D)

    inv_freq = 1.0 / (10000.0 ** (jnp.arange(0, HD, 2, dtype=jnp.float32) / HD))
    t = jnp.arange(S, dtype=jnp.float32)
    freqs = t[:, None] * inv_freq[None, :]
    emb = jnp.concatenate([freqs, freqs], axis=-1)
    cos = jnp.cos(emb)
    sin = jnp.sin(emb)

    q, k, v = pl.pallas_call(
        _rms_qkv_body,
        grid=(SB,),
        in_specs=[
            pl.BlockSpec((BS, D), lambda i: (i, 0)),
            pl.BlockSpec((1, D), lambda i: (0, 0)),
            pl.BlockSpec((D, D), lambda i: (0, 0)),
            pl.BlockSpec((D, D), lambda i: (0, 0)),
            pl.BlockSpec((D, D), lambda i: (0, 0)),
        ],
        out_specs=[pl.BlockSpec((BS, D), lambda i: (i, 0))] * 3,
        out_shape=[jax.ShapeDtypeStruct((S, D), jnp.float32)] * 3,
    )(x, w1, Wq, Wk, Wv)

    qh = q.reshape(S, H, HD).transpose(1, 0, 2)
    kh = k.reshape(S, H, HD).transpose(1, 0, 2)
    vh = v.reshape(S, H, HD).transpose(1, 0, 2)

    aoh = pl.pallas_call(
        _attn_body,
        grid=(H, S // BQ),
        in_specs=[
            pl.BlockSpec((1, BQ, HD), lambda h, i: (h, i, 0)),
            pl.BlockSpec((1, S, HD), lambda h, i: (h, 0, 0)),
            pl.BlockSpec((1, S, HD), lambda h, i: (h, 0, 0)),
            pl.BlockSpec((BQ, HD), lambda h, i: (i, 0)),
            pl.BlockSpec((BQ, HD), lambda h, i: (i, 0)),
            pl.BlockSpec((S, HD), lambda h, i: (0, 0)),
            pl.BlockSpec((S, HD), lambda h, i: (0, 0)),
        ],
        out_specs=pl.BlockSpec((1, BQ, HD), lambda h, i: (h, i, 0)),
        out_shape=jax.ShapeDtypeStruct((H, S, HD), jnp.float32),
    )(qh, kh, vh, cos, sin, cos, sin)
    ao = aoh.transpose(1, 0, 2).reshape(S, D)

    x2, h2, a1, a2, w1n, w2n, _us, _ps, lb = pl.pallas_call(
        _router_body,
        grid=(SB,),
        in_specs=[
            pl.BlockSpec((BS, D), lambda i: (i, 0)),
            pl.BlockSpec((BS, D), lambda i: (i, 0)),
            pl.BlockSpec((D, D), lambda i: (0, 0)),
            pl.BlockSpec((1, D), lambda i: (0, 0)),
            pl.BlockSpec((D, E), lambda i: (0, 0)),
        ],
        out_specs=[
            pl.BlockSpec((BS, D), lambda i: (i, 0)),
            pl.BlockSpec((BS, D), lambda i: (i, 0)),
            pl.BlockSpec((BS, 1), lambda i: (i, 0)),
            pl.BlockSpec((BS, 1), lambda i: (i, 0)),
            pl.BlockSpec((BS, 1), lambda i: (i, 0)),
            pl.BlockSpec((BS, 1), lambda i: (i, 0)),
            pl.BlockSpec((1, E), lambda i: (0, 0)),
            pl.BlockSpec((1, E), lambda i: (0, 0)),
            pl.BlockSpec((1, 1), lambda i: (0, 0)),
        ],
        out_shape=[
            jax.ShapeDtypeStruct((S, D), jnp.float32),
            jax.ShapeDtypeStruct((S, D), jnp.float32),
            jax.ShapeDtypeStruct((S, 1), jnp.int32),
            jax.ShapeDtypeStruct((S, 1), jnp.int32),
            jax.ShapeDtypeStruct((S, 1), jnp.float32),
            jax.ShapeDtypeStruct((S, 1), jnp.float32),
            jax.ShapeDtypeStruct((1, E), jnp.float32),
            jax.ShapeDtypeStruct((1, E), jnp.float32),
            jax.ShapeDtypeStruct((1, 1), jnp.float32),
        ],
    )(ao, x, Wo, w2, Wr)

    srcrow, posA, posB, bexp = _sc_plan(a1.reshape(S), a2.reshape(S))
    xg = _sc_gather(h2, srcrow)

    og = pl.pallas_call(
        _gmm_body,
        grid_spec=pltpu.PrefetchScalarGridSpec(
            num_scalar_prefetch=1,
            grid=(FB, NB),
            in_specs=[
                pl.BlockSpec((BLK, D), lambda f, nb, be: (nb, 0)),
                pl.BlockSpec((1, D, FT), lambda f, nb, be: (be[nb], 0, f)),
                pl.BlockSpec((1, D, FT), lambda f, nb, be: (be[nb], 0, f)),
                pl.BlockSpec((1, FT, D), lambda f, nb, be: (be[nb], f, 0)),
            ],
            out_specs=pl.BlockSpec((BLK, D), lambda f, nb, be: (nb, 0)),
            scratch_shapes=[pltpu.VMEM((PAD, D), jnp.float32)],
        ),
        out_shape=jax.ShapeDtypeStruct((PAD, D), jnp.float32),
    )(bexp[:NB], xg, Wg, Wu, Wd)

    gA, gB = _sc_combine_gather(og, posA, posB)

    out = pl.pallas_call(
        _combine_body,
        grid=(SB,),
        in_specs=[
            pl.BlockSpec((BS, D), lambda i: (i, 0)),
            pl.BlockSpec((BS, D), lambda i: (i, 0)),
            pl.BlockSpec((BS, D), lambda i: (i, 0)),
            pl.BlockSpec((BS, 1), lambda i: (i, 0)),
            pl.BlockSpec((BS, 1), lambda i: (i, 0)),
        ],
        out_specs=pl.BlockSpec((BS, D), lambda i: (i, 0)),
        out_shape=jax.ShapeDtypeStruct((S, D), jnp.float32),
    )(x2, gA, gB, w1n, w2n)

    return (out.reshape(B, S, D), lb.reshape(()))


# speedup vs baseline: 1.3620x; 1.0614x over previous
import functools

import jax
import jax.numpy as jnp
from jax import lax
from jax.experimental import pallas as pl
from jax.experimental.pallas import tpu as pltpu
from jax.experimental.pallas import tpu_sc as plsc

B, S, D, H, F, E, K = 1, 2048, 1024, 16, 4096, 8, 2
HD = D // H
EPS = 1e-6

BS = 256
BQ = 256
SB = S // BS

BLK = 128
NB = 40
PAD = NB * BLK
NBP = 48
FT = 1024
FB = F // FT

NW = 32
GCH = 40
LANE = 16


def _rms_qkv_body(x_ref, w_ref, wq_ref, wk_ref, wv_ref, q_ref, k_ref, v_ref):
    x = x_ref[...]
    h = x * lax.rsqrt(jnp.mean(x * x, axis=1, keepdims=True) + EPS) * w_ref[...]
    q = jnp.dot(h, wq_ref[...], preferred_element_type=jnp.float32)
    k = jnp.dot(h, wk_ref[...], preferred_element_type=jnp.float32)
    v = jnp.dot(h, wv_ref[...], preferred_element_type=jnp.float32)
    q_ref[...] = jnp.transpose(q.reshape(BS, H, HD), (1, 0, 2))
    k_ref[...] = jnp.transpose(k.reshape(BS, H, HD), (1, 0, 2))
    v_ref[...] = jnp.transpose(v.reshape(BS, H, HD), (1, 0, 2))


def _rope(x, cos, sin):
    x1 = x[:, : HD // 2]
    x2 = x[:, HD // 2:]
    rot = jnp.concatenate([-x2, x1], axis=1)
    return x * cos + rot * sin


def _attn_body(q_ref, k_ref, v_ref, cq_ref, sq_ref, ck_ref, sk_ref, o_ref):
    q = _rope(q_ref[0], cq_ref[...], sq_ref[...]) * (HD ** -0.5)
    k = _rope(k_ref[0], ck_ref[...], sk_ref[...])
    s = lax.dot_general(q, k, (((1,), (1,)), ((), ())),
                        preferred_element_type=jnp.float32)
    m = jnp.max(s, axis=1, keepdims=True)
    e = jnp.exp(s - m)
    num = jnp.dot(e, v_ref[0], preferred_element_type=jnp.float32)
    o_ref[0] = num / jnp.sum(e, axis=1, keepdims=True)


def _router_body(ao_ref, x_ref, wo_ref, w2_ref, wr_ref,
                 x2_ref, h2_ref, a1_ref, a2_ref, w1_ref, w2o_ref,
                 us_ref, ps_ref, lb_ref):
    sb = pl.program_id(0)
    ao = jnp.transpose(ao_ref[...], (1, 0, 2)).reshape(BS, D)
    x2 = x_ref[...] + jnp.dot(ao, wo_ref[...],
                              preferred_element_type=jnp.float32)
    x2_ref[...] = x2
    h2 = x2 * lax.rsqrt(jnp.mean(x2 * x2, axis=1, keepdims=True) + EPS) * w2_ref[...]
    h2_ref[...] = h2
    logits = jnp.dot(h2, wr_ref[...], preferred_element_type=jnp.float32)
    lmax = jnp.max(logits, axis=1, keepdims=True)
    el = jnp.exp(logits - lmax)
    probs = el / jnp.sum(el, axis=1, keepdims=True)
    ids = lax.broadcasted_iota(jnp.int32, (BS, E), 1)
    m1 = jnp.max(probs, axis=1, keepdims=True)
    a1 = jnp.min(jnp.where(probs == m1, ids, E), axis=1, keepdims=True)
    oh1 = (ids == a1).astype(jnp.float32)
    probs2 = jnp.where(ids == a1, -1.0, probs)
    m2 = jnp.max(probs2, axis=1, keepdims=True)
    a2 = jnp.min(jnp.where(probs2 == m2, ids, E), axis=1, keepdims=True)
    oh2 = (ids == a2).astype(jnp.float32)
    wsum = m1 + m2
    a1_ref[...] = a1
    a2_ref[...] = a2
    w1_ref[...] = m1 / wsum
    w2o_ref[...] = m2 / wsum

    @pl.when(sb == 0)
    def _init():
        us_ref[...] = jnp.zeros_like(us_ref)
        ps_ref[...] = jnp.zeros_like(ps_ref)
        lb_ref[...] = jnp.zeros_like(lb_ref)

    us_ref[...] += jnp.sum(oh1 + oh2, axis=0, keepdims=True)
    ps_ref[...] += jnp.sum(probs, axis=0, keepdims=True)

    @pl.when(sb == SB - 1)
    def _fin():
        lb_ref[...] = jnp.sum(us_ref[...] * ps_ref[...], axis=1, keepdims=True) \
            * (float(E) / (S * float(S)))


_SC_MESH = plsc.VectorSubcoreMesh(core_axis_name="c", subcore_axis_name="s")
NCH = S // LANE


@functools.partial(
    pl.kernel,
    mesh=_SC_MESH,
    compiler_params=pltpu.CompilerParams(needs_layout_passes=False),
    out_type=[
        jax.ShapeDtypeStruct((PAD,), jnp.int32),
        jax.ShapeDtypeStruct((S,), jnp.int32),
        jax.ShapeDtypeStruct((S,), jnp.int32),
        jax.ShapeDtypeStruct((NBP,), jnp.int32),
    ],
    scratch_types=[
        pltpu.VMEM((S,), jnp.int32),
        pltpu.VMEM((S,), jnp.int32),
        pltpu.VMEM((S,), jnp.int32),
        pltpu.VMEM((S,), jnp.int32),
        pltpu.VMEM((PAD,), jnp.int32),
        pltpu.VMEM((S,), jnp.int32),
        pltpu.VMEM((S,), jnp.int32),
        pltpu.VMEM((NBP,), jnp.int32),
        pltpu.VMEM((LANE,), jnp.int32),
        pltpu.VMEM((LANE,), jnp.int32),
    ],
)
def _sc_plan(eiA_hbm, eiB_hbm, srcrow_hbm, posA_hbm, posB_hbm, bexp_hbm,
             eA_v, eB_v, rankA_v, rankB_v, srcrow_v, posA_v, posB_v, bexp_v,
             cnt_v, off_v):
    wid = lax.axis_index("s") * 2 + lax.axis_index("c")

    @pl.when(wid == 0)
    def _():
        pltpu.sync_copy(eiA_hbm, eA_v)
        pltpu.sync_copy(eiB_hbm, eB_v)
        lane = lax.iota(jnp.int32, LANE)
        cnt_v[...] = jnp.zeros((LANE,), jnp.int32)

        def rank_pass(src_v, dst_v):
            def body(c, _):
                ch = src_v[pl.ds(c * LANE, LANE)]
                cnt = cnt_v[...]
                rank = jnp.zeros((LANE,), jnp.int32)
                for e in range(E):
                    m = ch == e
                    mi = jnp.where(m, 1, 0)
                    cs = plsc.cumsum(mi)
                    cnt_e = jnp.sum(jnp.where(lane == e, cnt, 0))
                    rank = jnp.where(m, cnt_e + cs - 1, rank)
                    tote = jnp.sum(mi)
                    cnt = jnp.where(lane == e, cnt + tote, cnt)
                dst_v[pl.ds(c * LANE, LANE)] = rank
                cnt_v[...] = cnt
                return 0
            lax.fori_loop(0, NCH, body, 0)

        rank_pass(eA_v, rankA_v)
        rank_pass(eB_v, rankB_v)

        cnt = cnt_v[...]
        blocks = lax.shift_right_logical(cnt + (BLK - 1), 7)
        cumblk = plsc.cumsum(blocks)
        off_v[...] = (cumblk - blocks) * BLK

        def zero_body(i, _):
            srcrow_v[pl.ds(i * LANE, LANE)] = jnp.zeros((LANE,), jnp.int32)
            return 0
        lax.fori_loop(0, PAD // LANE, zero_body, 0)

        def pos_pass(src_v, rank_v, pos_v):
            def body(c, _):
                ch = src_v[pl.ds(c * LANE, LANE)]
                offv = off_v[...]
                off = jnp.zeros((LANE,), jnp.int32)
                for e in range(E):
                    off_e = jnp.sum(jnp.where(lane == e, offv, 0))
                    off = jnp.where(ch == e, off_e, off)
                pos = off + rank_v[pl.ds(c * LANE, LANE)]
                pos_v[pl.ds(c * LANE, LANE)] = pos
                tok = lane + c * LANE
                plsc.store_scatter(srcrow_v, [pos], tok)
                return 0
            lax.fori_loop(0, NCH, body, 0)

        pos_pass(eA_v, rankA_v, posA_v)
        pos_pass(eB_v, rankB_v, posB_v)

        for cc in range(NBP // LANE):
            nb = lane + cc * LANE
            be = jnp.zeros((LANE,), jnp.int32)
            for e in range(E - 1):
                ce = jnp.sum(jnp.where(lane == e, cumblk, 0))
                be = be + jnp.where(nb >= ce, 1, 0)
            bexp_v[pl.ds(cc * LANE, LANE)] = be

        pltpu.sync_copy(srcrow_v, srcrow_hbm)
        pltpu.sync_copy(posA_v, posA_hbm)
        pltpu.sync_copy(posB_v, posB_hbm)
        pltpu.sync_copy(bexp_v, bexp_hbm)


RPW = PAD // NW
GNC = RPW // GCH


@functools.partial(
    pl.kernel,
    mesh=_SC_MESH,
    compiler_params=pltpu.CompilerParams(needs_layout_passes=False),
    out_type=jax.ShapeDtypeStruct((PAD, D), jnp.float32),
    scratch_types=[
        pltpu.VMEM((RPW,), jnp.int32),
        pltpu.VMEM((GCH, D), jnp.float32),
        pltpu.VMEM((GCH, D), jnp.float32),
        pltpu.SemaphoreType.DMA,
        pltpu.SemaphoreType.DMA,
    ],
)
def _sc_gather(h2_hbm, srcrow_hbm, xg_hbm, idx_v, buf0, buf1, sem0, sem1):
    wid = lax.axis_index("s") * 2 + lax.axis_index("c")
    base = wid * RPW
    pltpu.sync_copy(srcrow_hbm.at[pl.ds(base, RPW)], idx_v)
    bufs = (buf0, buf1)
    sems = (sem0, sem1)
    cps = []
    for j in range(GNC):
        if j >= 2:
            cps[j - 2].wait()
            pltpu.sync_copy(bufs[j % 2], xg_hbm.at[pl.ds(base + (j - 2) * GCH, GCH)])
        cps.append(pltpu.async_copy(
            h2_hbm.at[idx_v.at[pl.ds(j * GCH, GCH)]], bufs[j % 2], sems[j % 2]))
    for j in range(max(0, GNC - 2), GNC):
        cps[j].wait()
        pltpu.sync_copy(bufs[j % 2], xg_hbm.at[pl.ds(base + j * GCH, GCH)])


CCH = S // NW


CCC = CCH // 2


@functools.partial(
    pl.kernel,
    mesh=_SC_MESH,
    compiler_params=pltpu.CompilerParams(needs_layout_passes=False),
    out_type=[
        jax.ShapeDtypeStruct((S, D), jnp.float32),
        jax.ShapeDtypeStruct((S, D), jnp.float32),
    ],
    scratch_types=[
        pltpu.VMEM((CCH,), jnp.int32),
        pltpu.VMEM((CCH,), jnp.int32),
        pltpu.VMEM((CCC, D), jnp.float32),
        pltpu.VMEM((CCC, D), jnp.float32),
        pltpu.SemaphoreType.DMA,
        pltpu.SemaphoreType.DMA,
    ],
)
def _sc_combine_gather(og_hbm, posA_hbm, posB_hbm, gA_hbm, gB_hbm,
                       idxA_v, idxB_v, buf0, buf1, sem0, sem1):
    wid = lax.axis_index("s") * 2 + lax.axis_index("c")
    base = wid * CCH
    pltpu.sync_copy(posA_hbm.at[pl.ds(base, CCH)], idxA_v)
    pltpu.sync_copy(posB_hbm.at[pl.ds(base, CCH)], idxB_v)
    bufs = (buf0, buf1)
    sems = (sem0, sem1)
    plan = [
        (idxA_v, gA_hbm, 0), (idxA_v, gA_hbm, 1),
        (idxB_v, gB_hbm, 0), (idxB_v, gB_hbm, 1),
    ]
    cps = []
    for j, (idx, dst, half) in enumerate(plan):
        if j >= 2:
            pidx, pdst, phalf = plan[j - 2]
            cps[j - 2].wait()
            pltpu.sync_copy(bufs[(j - 2) % 2],
                            pdst.at[pl.ds(base + phalf * CCC, CCC)])
        cps.append(pltpu.async_copy(
            og_hbm.at[idx.at[pl.ds(half * CCC, CCC)]], bufs[j % 2], sems[j % 2]))
    for j in range(2, 4):
        pidx, pdst, phalf = plan[j]
        cps[j].wait()
        pltpu.sync_copy(bufs[j % 2], pdst.at[pl.ds(base + phalf * CCC, CCC)])


def _gmm_body(bexp_ref, xg_ref, wg_ref, wu_ref, wd_ref, og_ref, acc_ref):
    f = pl.program_id(0)
    nb = pl.program_id(1)
    xb = xg_ref[...]
    g = jax.nn.silu(jnp.dot(xb, wg_ref[0], preferred_element_type=jnp.float32))
    u = jnp.dot(xb, wu_ref[0], preferred_element_type=jnp.float32)
    contrib = jnp.dot(g * u, wd_ref[0], preferred_element_type=jnp.float32)
    sl = pl.ds(nb * BLK, BLK)

    @pl.when(f == 0)
    def _first():
        acc_ref[sl, :] = contrib

    @pl.when(f > 0)
    def _rest():
        acc_ref[sl, :] += contrib

    @pl.when(f == FB - 1)
    def _write():
        og_ref[...] = acc_ref[sl, :]


def _combine_body(x2_ref, ga_ref, gb_ref, w1_ref, w2_ref, o_ref):
    o_ref[...] = x2_ref[...] + w1_ref[...] * ga_ref[...] + w2_ref[...] * gb_ref[...]


def kernel(hidden_states, ln1_w, ln2_w, Wq, Wk, Wv, Wo, Wr, Wg, Wu, Wd):
    x = hidden_states.reshape(S, D)
    w1 = ln1_w.reshape(1, D)
    w2 = ln2_w.reshape(1, D)

    inv_freq = 1.0 / (10000.0 ** (jnp.arange(0, HD, 2, dtype=jnp.float32) / HD))
    t = jnp.arange(S, dtype=jnp.float32)
    freqs = t[:, None] * inv_freq[None, :]
    emb = jnp.concatenate([freqs, freqs], axis=-1)
    cos = jnp.cos(emb)
    sin = jnp.sin(emb)

    qh, kh, vh = pl.pallas_call(
        _rms_qkv_body,
        grid=(SB,),
        in_specs=[
            pl.BlockSpec((BS, D), lambda i: (i, 0)),
            pl.BlockSpec((1, D), lambda i: (0, 0)),
            pl.BlockSpec((D, D), lambda i: (0, 0)),
            pl.BlockSpec((D, D), lambda i: (0, 0)),
            pl.BlockSpec((D, D), lambda i: (0, 0)),
        ],
        out_specs=[pl.BlockSpec((H, BS, HD), lambda i: (0, i, 0))] * 3,
        out_shape=[jax.ShapeDtypeStruct((H, S, HD), jnp.float32)] * 3,
    )(x, w1, Wq, Wk, Wv)

    aoh = pl.pallas_call(
        _attn_body,
        grid=(H, S // BQ),
        in_specs=[
            pl.BlockSpec((1, BQ, HD), lambda h, i: (h, i, 0)),
            pl.BlockSpec((1, S, HD), lambda h, i: (h, 0, 0)),
            pl.BlockSpec((1, S, HD), lambda h, i: (h, 0, 0)),
            pl.BlockSpec((BQ, HD), lambda h, i: (i, 0)),
            pl.BlockSpec((BQ, HD), lambda h, i: (i, 0)),
            pl.BlockSpec((S, HD), lambda h, i: (0, 0)),
            pl.BlockSpec((S, HD), lambda h, i: (0, 0)),
        ],
        out_specs=pl.BlockSpec((1, BQ, HD), lambda h, i: (h, i, 0)),
        out_shape=jax.ShapeDtypeStruct((H, S, HD), jnp.float32),
    )(qh, kh, vh, cos, sin, cos, sin)

    x2, h2, a1, a2, w1n, w2n, _us, _ps, lb = pl.pallas_call(
        _router_body,
        grid=(SB,),
        in_specs=[
            pl.BlockSpec((H, BS, HD), lambda i: (0, i, 0)),
            pl.BlockSpec((BS, D), lambda i: (i, 0)),
            pl.BlockSpec((D, D), lambda i: (0, 0)),
            pl.BlockSpec((1, D), lambda i: (0, 0)),
            pl.BlockSpec((D, E), lambda i: (0, 0)),
        ],
        out_specs=[
            pl.BlockSpec((BS, D), lambda i: (i, 0)),
            pl.BlockSpec((BS, D), lambda i: (i, 0)),
            pl.BlockSpec((BS, 1), lambda i: (i, 0)),
            pl.BlockSpec((BS, 1), lambda i: (i, 0)),
            pl.BlockSpec((BS, 1), lambda i: (i, 0)),
            pl.BlockSpec((BS, 1), lambda i: (i, 0)),
            pl.BlockSpec((1, E), lambda i: (0, 0)),
            pl.BlockSpec((1, E), lambda i: (0, 0)),
            pl.BlockSpec((1, 1), lambda i: (0, 0)),
        ],
        out_shape=[
            jax.ShapeDtypeStruct((S, D), jnp.float32),
            jax.ShapeDtypeStruct((S, D), jnp.float32),
            jax.ShapeDtypeStruct((S, 1), jnp.int32),
            jax.ShapeDtypeStruct((S, 1), jnp.int32),
            jax.ShapeDtypeStruct((S, 1), jnp.float32),
            jax.ShapeDtypeStruct((S, 1), jnp.float32),
            jax.ShapeDtypeStruct((1, E), jnp.float32),
            jax.ShapeDtypeStruct((1, E), jnp.float32),
            jax.ShapeDtypeStruct((1, 1), jnp.float32),
        ],
    )(aoh, x, Wo, w2, Wr)

    srcrow, posA, posB, bexp = _sc_plan(a1.reshape(S), a2.reshape(S))
    xg = _sc_gather(h2, srcrow)

    og = pl.pallas_call(
        _gmm_body,
        grid_spec=pltpu.PrefetchScalarGridSpec(
            num_scalar_prefetch=1,
            grid=(FB, NB),
            in_specs=[
                pl.BlockSpec((BLK, D), lambda f, nb, be: (nb, 0)),
                pl.BlockSpec((1, D, FT), lambda f, nb, be: (be[nb], 0, f)),
                pl.BlockSpec((1, D, FT), lambda f, nb, be: (be[nb], 0, f)),
                pl.BlockSpec((1, FT, D), lambda f, nb, be: (be[nb], f, 0)),
            ],
            out_specs=pl.BlockSpec((BLK, D), lambda f, nb, be: (nb, 0)),
            scratch_shapes=[pltpu.VMEM((PAD, D), jnp.float32)],
        ),
        out_shape=jax.ShapeDtypeStruct((PAD, D), jnp.float32),
    )(bexp[:NB], xg, Wg, Wu, Wd)

    gA, gB = _sc_combine_gather(og, posA, posB)

    out = pl.pallas_call(
        _combine_body,
        grid=(SB,),
        in_specs=[
            pl.BlockSpec((BS, D), lambda i: (i, 0)),
            pl.BlockSpec((BS, D), lambda i: (i, 0)),
            pl.BlockSpec((BS, D), lambda i: (i, 0)),
            pl.BlockSpec((BS, 1), lambda i: (i, 0)),
            pl.BlockSpec((BS, 1), lambda i: (i, 0)),
        ],
        out_specs=pl.BlockSpec((BS, D), lambda i: (i, 0)),
        out_shape=jax.ShapeDtypeStruct((S, D), jnp.float32),
    )(x2, gA, gB, w1n, w2n)

    return (out.reshape(B, S, D), lb.reshape(()))
